# Initial kernel scaffold; baseline (speedup 1.0000x reference)
import jax
import jax.numpy as jnp
from jax.experimental import pallas as pl


def kernel(input, params, sampled_ints):
    raise NotImplementedError("write your pallas kernel here")


# speedup vs baseline: 5.3943x; 5.3943x over previous
import functools
import jax
import jax.numpy as jnp
from jax import lax
from jax.experimental import pallas as pl
from jax.experimental.pallas import tpu as pltpu
from jax.experimental.pallas import tpu_sc as plsc

EPS = 1e-6
OUT_N = 10000
IN_N = 10000
KN = 160000
PN = 8
DN = 128
DH = DN // 2

N_SUB = 16
EDGES = KN * PN
E_PER_SUB = EDGES // N_SUB
CHUNK = 80
R_CH = 25
N_CH = E_PER_SUB // CHUNK
N_REF = N_CH // R_CH

KB = 1250
BK = 50
GRID = KB // BK


def _hyper_body(p0_ref, p1_ref, ps_ref, pv_ref, sr_ref, sc_ref,
                rows_ref, cols_ref, vals_ref):
    p0 = p0_ref[...]
    p1 = p1_ref[...]
    m0 = jax.nn.sigmoid(p0) * (OUT_N - 1.0)
    m1 = jax.nn.sigmoid(p1) * (IN_N - 1.0)
    sg = jnp.logaddexp(ps_ref[...] + 2.0, 0.0) + EPS
    s0 = sg * (OUT_N * 0.2)
    s1 = sg * (IN_N * 0.2)
    val = pv_ref[...]

    f0 = jnp.floor(m0)
    c0 = jnp.ceil(m0)
    f1 = jnp.floor(m1)
    c1 = jnp.ceil(m1)

    r_fl = [f0, f0, c0, c0,
            sr_ref[0].astype(jnp.float32), sr_ref[1].astype(jnp.float32),
            sr_ref[2].astype(jnp.float32), sr_ref[3].astype(jnp.float32)]
    c_fl = [f1, c1, f1, c1,
            sc_ref[0].astype(jnp.float32), sc_ref[1].astype(jnp.float32),
            sc_ref[2].astype(jnp.float32), sc_ref[3].astype(jnp.float32)]

    props = []
    for p in range(PN):
        d0 = (r_fl[p] - m0) / s0
        d1 = (c_fl[p] - m1) / s1
        props.append(jnp.exp(-0.5 * (d0 * d0 + d1 * d1)))
    denom = props[0] + EPS
    for p in range(1, PN):
        denom = denom + (props[p] + EPS)
    for p in range(PN):
        v_p = val * props[p] / denom
        rows_ref[p] = jnp.clip(r_fl[p].astype(jnp.int32), 0, OUT_N - 1)
        cols_ref[p] = jnp.clip(c_fl[p].astype(jnp.int32), 0, IN_N - 1)
        vals_ref[p] = v_p


def _hyper(params, sampled_ints):
    p0 = params[:, 0].reshape(KB, 128)
    p1 = params[:, 1].reshape(KB, 128)
    ps = params[:, 2].reshape(KB, 128)
    pv = params[:, 3].reshape(KB, 128)
    sr = sampled_ints[:, :, 0].T.reshape(4, KB, 128)
    sc = sampled_ints[:, :, 1].T.reshape(4, KB, 128)

    rows, cols, vals = pl.pallas_call(
        _hyper_body,
        out_shape=[
            jax.ShapeDtypeStruct((PN, KB, 128), jnp.int32),
            jax.ShapeDtypeStruct((PN, KB, 128), jnp.int32),
            jax.ShapeDtypeStruct((PN, KB, 128), jnp.float32),
        ],
    )(p0, p1, ps, pv, sr, sc)
    return rows, cols, vals


def _sc_body(tbl_hbm, rows_hbm, cols_hbm, vals_hbm, out_hbm,
             rows_v, cols_v, vals_v, gbuf, zbuf, acc, sem):
    c = lax.axis_index("c")
    s = lax.axis_index("s")

    def _z(i, _):
        zbuf[i // 4, pl.ds((i % 4) * 16, 16)] = jnp.zeros((16,), jnp.float32)
        return 0
    lax.fori_loop(0, 500, _z, 0)
    for t in range(5):
        pltpu.sync_copy(zbuf, acc.at[pl.ds(s * 625 + t * 125, 125)])
    plsc.subcore_barrier()

    def _refill(r, _):
        pltpu.sync_copy(rows_hbm.at[s, pl.ds(r * R_CH, R_CH)], rows_v)
        pltpu.sync_copy(cols_hbm.at[c, s, pl.ds(r * R_CH, R_CH)], cols_v)
        pltpu.sync_copy(vals_hbm.at[s, pl.ds(r * R_CH, R_CH)], vals_v)

        def _chunk(j, _):
            pltpu.async_copy(tbl_hbm.at[cols_v.at[j]], gbuf, sem).wait()

            def _scale(e, _):
                vb = plsc.load_gather(
                    vals_v,
                    [jnp.full((16,), j, jnp.int32), jnp.full((16,), e, jnp.int32)],
                )
                for g in range(4):
                    gbuf[e, pl.ds(g * 16, 16)] = gbuf[e, pl.ds(g * 16, 16)] * vb
                return 0
            lax.fori_loop(0, CHUNK, _scale, 0)
            pltpu.sync_copy(gbuf, acc.at[rows_v.at[j]], add=True)
            return 0
        lax.fori_loop(0, R_CH, _chunk, 0)
        return 0
    lax.fori_loop(0, N_REF, _refill, 0)

    plsc.subcore_barrier()
    for t in range(5):
        base = s * 625 + t * 125
        pltpu.sync_copy(acc.at[pl.ds(base, 125)],
                        out_hbm.at[pl.ds(base, 125), pl.ds(c * DH, DH)])


@functools.partial(jax.jit, donate_argnums=())
def kernel(input, params, sampled_ints):
    rows8, cols8, vals8 = _hyper(params, sampled_ints)
    rows16 = rows8.reshape(N_SUB, N_CH, CHUNK)
    cols16 = cols8.reshape(N_SUB, N_CH, CHUNK)
    cols2 = jnp.stack([cols16, cols16 + IN_N])
    vals16 = vals8.reshape(N_SUB, N_CH, CHUNK)
    tbl = jnp.concatenate([input[:, :DH], input[:, DH:]], axis=0)

    sc_call = pl.kernel(
        _sc_body,
        out_type=jax.ShapeDtypeStruct((OUT_N, DN), jnp.float32),
        mesh=plsc.VectorSubcoreMesh(core_axis_name="c", subcore_axis_name="s"),
        compiler_params=pltpu.CompilerParams(use_tc_tiling_on_sc=False,
                                            needs_layout_passes=False),
        scratch_types=[
            pltpu.VMEM((R_CH, CHUNK), jnp.int32),
            pltpu.VMEM((R_CH, CHUNK), jnp.int32),
            pltpu.VMEM((R_CH, CHUNK), jnp.float32),
            pltpu.VMEM((CHUNK, DH), jnp.float32),
            pltpu.VMEM((125, DH), jnp.float32),
            pltpu.VMEM_SHARED((OUT_N, DH), jnp.float32),
            pltpu.SemaphoreType.DMA,
        ],
    )
    return sc_call(tbl, rows16, cols2, vals16)
